# Initial kernel scaffold; baseline (speedup 1.0000x reference)
import jax
import jax.numpy as jnp
from jax.experimental import pallas as pl


def kernel(k_cache, v_cache, input_pos, k_val, v_val):
    raise NotImplementedError("write your pallas kernel here")


# speedup vs baseline: 1.0112x; 1.0112x over previous
import jax
import jax.numpy as jnp
from jax.experimental import pallas as pl
from jax.experimental.pallas import tpu as pltpu

_B, _H, _S, _D = 8, 16, 2048, 128
_L = 16


def _body(pos_ref, kc_ref, vc_ref, kv_ref, vv_ref, ko_ref, vo_ref):
    ko_ref[...] = kc_ref[...]
    vo_ref[...] = vc_ref[...]
    for l in range(_L):
        p = pos_ref[l]
        ko_ref[0, 0, pl.ds(p, 1), :] = kv_ref[0, 0, pl.ds(l, 1), :]
        vo_ref[0, 0, pl.ds(p, 1), :] = vv_ref[0, 0, pl.ds(l, 1), :]


def kernel(k_cache, v_cache, input_pos, k_val, v_val):
    cache_spec = pl.BlockSpec((1, 1, _S, _D), lambda i, j, pos: (i, j, 0, 0))
    val_spec = pl.BlockSpec((1, 1, _L, _D), lambda i, j, pos: (i, j, 0, 0))
    out = pl.pallas_call(
        _body,
        grid_spec=pltpu.PrefetchScalarGridSpec(
            num_scalar_prefetch=1,
            grid=(_B, _H),
            in_specs=[cache_spec, cache_spec, val_spec, val_spec],
            out_specs=[cache_spec, cache_spec],
        ),
        out_shape=[jax.ShapeDtypeStruct((_B, _H, _S, _D), jnp.float32)] * 2,
        compiler_params=pltpu.CompilerParams(
            dimension_semantics=("arbitrary", "arbitrary"),
        ),
    )(input_pos, k_cache, v_cache, k_val, v_val)
    return (out[0], out[1])
